# speedup vs baseline: 1.9266x; 1.9266x over previous
import jax
import jax.numpy as jnp
from jax.experimental import pallas as pl
from jax.experimental.pallas import tpu as pltpu

_HID = 512
_K = 1024
_COMMIT = 0.25
_TM = 512


def _enc_pre_kernel(obs_ref, act_ref, wo_ref, wa_ref, b1_ref, wih_ref, bih_ref,
                    out_ref):
    x = jnp.dot(obs_ref[...], wo_ref[...], preferred_element_type=jnp.float32)
    x = x + jnp.dot(act_ref[...], wa_ref[...],
                    preferred_element_type=jnp.float32)
    x = jax.nn.relu(x + b1_ref[...])
    out_ref[...] = jnp.dot(x, wih_ref[...],
                           preferred_element_type=jnp.float32) + bih_ref[...]


def _mm_bias_kernel(x_ref, w_ref, b_ref, out_ref):
    out_ref[...] = jnp.dot(x_ref[...], w_ref[...],
                           preferred_element_type=jnp.float32) + b_ref[...]


def _dec_pre_kernel(obs_ref, w1_ref, b1_ref, wih_ref, bih_ref, out_ref):
    x = jax.nn.relu(jnp.dot(obs_ref[...], w1_ref[...],
                            preferred_element_type=jnp.float32) + b1_ref[...])
    out_ref[...] = jnp.dot(x, wih_ref[...],
                           preferred_element_type=jnp.float32) + bih_ref[...]


def _gru_step(gi, h, whh_ref, bhh_ref):
    gh = jnp.dot(h, whh_ref[...],
                 preferred_element_type=jnp.float32) + bhh_ref[...]
    i_r, i_z, i_n = gi[:, :_HID], gi[:, _HID:2 * _HID], gi[:, 2 * _HID:]
    h_r, h_z, h_n = gh[:, :_HID], gh[:, _HID:2 * _HID], gh[:, 2 * _HID:]
    r = jax.nn.sigmoid(i_r + h_r)
    z = jax.nn.sigmoid(i_z + h_z)
    n = jnp.tanh(i_n + r * h_n)
    return (1.0 - z) * n + z * h


def _gru_scan_kernel(gi_ref, whh_ref, bhh_ref, out_ref, h_ref):
    t = pl.program_id(0)

    @pl.when(t == 0)
    def _():
        h_ref[...] = jnp.zeros_like(h_ref)

    h_new = _gru_step(gi_ref[0], h_ref[...], whh_ref, bhh_ref)
    h_ref[...] = h_new
    out_ref[0] = h_new


def _gru_last_kernel(gi_ref, whh_ref, bhh_ref, out_ref, h_ref):
    t = pl.program_id(0)

    @pl.when(t == 0)
    def _():
        h_ref[...] = jnp.zeros_like(h_ref)

    h_new = _gru_step(gi_ref[0], h_ref[...], whh_ref, bhh_ref)
    h_ref[...] = h_new

    @pl.when(t == pl.num_programs(0) - 1)
    def _():
        out_ref[...] = h_new


def _vq_kernel(h_ref, wz_ref, bz_ref, cbT_ref, cb_ref, c2h_ref, c2hb_ref,
               idx_ref, loss_ref, dh0_ref):
    z = jnp.dot(h_ref[...], wz_ref[...],
                preferred_element_type=jnp.float32) + bz_ref[...]
    scores = jnp.dot(z, cbT_ref[...], preferred_element_type=jnp.float32)
    csq = jnp.sum(cbT_ref[...] * cbT_ref[...], axis=0, keepdims=True)
    zsq = jnp.sum(z * z, axis=1, keepdims=True)
    d = zsq + csq - 2.0 * scores
    m = jnp.min(d, axis=1, keepdims=True)
    iota = jax.lax.broadcasted_iota(jnp.int32, d.shape, 1)
    idx = jnp.min(jnp.where(d == m, iota, _K), axis=1, keepdims=True)
    idx_ref[...] = jnp.broadcast_to(idx, idx_ref.shape)
    onehot = (iota == idx).astype(jnp.float32)
    z_q = jnp.dot(onehot, cb_ref[...], preferred_element_type=jnp.float32)
    diff = z_q - z
    loss_ref[...] = ((1.0 + _COMMIT) * jnp.mean(diff * diff)).reshape(1, 1)
    dh0_ref[...] = jnp.tanh(
        jnp.dot(z_q, c2h_ref[...], preferred_element_type=jnp.float32)
        + c2hb_ref[...])


def _dec_scan_kernel(gi_ref, act_ref, dh0_ref, whh_ref, bhh_ref, whd_ref,
                     bhd_ref, logits_ref, loss_ref, h_ref, acc_ref):
    t = pl.program_id(0)

    @pl.when(t == 0)
    def _():
        h_ref[...] = dh0_ref[...]
        acc_ref[0, 0] = 0.0

    h_new = _gru_step(gi_ref[0], h_ref[...], whh_ref, bhh_ref)
    h_ref[...] = h_new
    logits = jnp.dot(h_new, whd_ref[...],
                     preferred_element_type=jnp.float32) + bhd_ref[...]
    logits_ref[0] = logits

    a = act_ref[0]
    na = a.shape[1]
    ai = jax.lax.broadcasted_iota(jnp.int32, a.shape, 1)
    amax = jnp.max(a, axis=1, keepdims=True)
    tgt = jnp.min(jnp.where(a == amax, ai, na), axis=1, keepdims=True)
    lmax = jnp.max(logits, axis=1, keepdims=True)
    lse = lmax + jnp.log(
        jnp.sum(jnp.exp(logits - lmax), axis=1, keepdims=True))
    li = jax.lax.broadcasted_iota(jnp.int32, logits.shape, 1)
    tgt_logp = jnp.sum(jnp.where(li == tgt, logits - lse, 0.0), axis=1)
    acc_ref[0, 0] += jnp.sum(tgt_logp)

    @pl.when(t == pl.num_programs(0) - 1)
    def _():
        denom = h_ref.shape[0] * pl.num_programs(0)
        loss_ref[...] = (-acc_ref[0, 0] / denom).reshape(1, 1)


_SEQ = pltpu.CompilerParams(dimension_semantics=("arbitrary",))
_PAR = pltpu.CompilerParams(dimension_semantics=("parallel",))


def _gru_scan(gi3, whhT, bhh, full_outputs):
    S, B, H3 = gi3.shape
    H = H3 // 3
    in_specs = [
        pl.BlockSpec((1, B, H3), lambda t: (t, 0, 0)),
        pl.BlockSpec((H, H3), lambda t: (0, 0)),
        pl.BlockSpec((1, H3), lambda t: (0, 0)),
    ]
    if full_outputs:
        return pl.pallas_call(
            _gru_scan_kernel,
            grid=(S,),
            in_specs=in_specs,
            out_specs=pl.BlockSpec((1, B, H), lambda t: (t, 0, 0)),
            out_shape=jax.ShapeDtypeStruct((S, B, H), jnp.float32),
            scratch_shapes=[pltpu.VMEM((B, H), jnp.float32)],
            compiler_params=_SEQ,
        )(gi3, whhT, bhh)
    return pl.pallas_call(
        _gru_last_kernel,
        grid=(S,),
        in_specs=in_specs,
        out_specs=pl.BlockSpec((B, H), lambda t: (0, 0)),
        out_shape=jax.ShapeDtypeStruct((B, H), jnp.float32),
        scratch_shapes=[pltpu.VMEM((B, H), jnp.float32)],
        compiler_params=_SEQ,
    )(gi3, whhT, bhh)


def kernel(obs_seq, act_seq, enc_in_W, enc_in_b, enc_W_ih0, enc_W_hh0,
           enc_b_ih0, enc_b_hh0, enc_W_ih1, enc_W_hh1, enc_b_ih1, enc_b_hh1,
           enc_out_W, enc_out_b, codebook, c2h_W, c2h_b, dec_in_W, dec_in_b,
           dec_W_ih, dec_W_hh, dec_b_ih, dec_b_hh, head_W, head_b):
    B, S, OBS = obs_seq.shape
    A = act_seq.shape[-1]
    H = _HID
    M = S * B

    obsT = jnp.swapaxes(obs_seq, 0, 1).reshape(M, OBS)
    actT3 = jnp.swapaxes(act_seq, 0, 1)

    grid_m = M // _TM

    gi0 = pl.pallas_call(
        _enc_pre_kernel,
        grid=(grid_m,),
        in_specs=[
            pl.BlockSpec((_TM, OBS), lambda i: (i, 0)),
            pl.BlockSpec((_TM, A), lambda i: (i, 0)),
            pl.BlockSpec((OBS, H), lambda i: (0, 0)),
            pl.BlockSpec((A, H), lambda i: (0, 0)),
            pl.BlockSpec((1, H), lambda i: (0, 0)),
            pl.BlockSpec((H, 3 * H), lambda i: (0, 0)),
            pl.BlockSpec((1, 3 * H), lambda i: (0, 0)),
        ],
        out_specs=pl.BlockSpec((_TM, 3 * H), lambda i: (i, 0)),
        out_shape=jax.ShapeDtypeStruct((M, 3 * H), jnp.float32),
        compiler_params=_PAR,
    )(obsT, actT3.reshape(M, A), enc_in_W[:, :OBS].T, enc_in_W[:, OBS:].T,
      enc_in_b.reshape(1, H), enc_W_ih0.T, enc_b_ih0.reshape(1, 3 * H))

    out0 = _gru_scan(gi0.reshape(S, B, 3 * H), enc_W_hh0.T,
                     enc_b_hh0.reshape(1, 3 * H), full_outputs=True)

    gi1 = pl.pallas_call(
        _mm_bias_kernel,
        grid=(grid_m,),
        in_specs=[
            pl.BlockSpec((_TM, H), lambda i: (i, 0)),
            pl.BlockSpec((H, 3 * H), lambda i: (0, 0)),
            pl.BlockSpec((1, 3 * H), lambda i: (0, 0)),
        ],
        out_specs=pl.BlockSpec((_TM, 3 * H), lambda i: (i, 0)),
        out_shape=jax.ShapeDtypeStruct((M, 3 * H), jnp.float32),
        compiler_params=_PAR,
    )(out0.reshape(M, H), enc_W_ih1.T, enc_b_ih1.reshape(1, 3 * H))

    h_last = _gru_scan(gi1.reshape(S, B, 3 * H), enc_W_hh1.T,
                       enc_b_hh1.reshape(1, 3 * H), full_outputs=False)

    EMB = codebook.shape[1]
    idx8, vq_loss11, dh0 = pl.pallas_call(
        _vq_kernel,
        out_shape=[
            jax.ShapeDtypeStruct((B, 8), jnp.int32),
            jax.ShapeDtypeStruct((1, 1), jnp.float32),
            jax.ShapeDtypeStruct((B, H), jnp.float32),
        ],
    )(h_last, enc_out_W.T, enc_out_b.reshape(1, EMB), codebook.T, codebook,
      c2h_W.T, c2h_b.reshape(1, H))

    gid = pl.pallas_call(
        _dec_pre_kernel,
        grid=(grid_m,),
        in_specs=[
            pl.BlockSpec((_TM, OBS), lambda i: (i, 0)),
            pl.BlockSpec((OBS, H), lambda i: (0, 0)),
            pl.BlockSpec((1, H), lambda i: (0, 0)),
            pl.BlockSpec((H, 3 * H), lambda i: (0, 0)),
            pl.BlockSpec((1, 3 * H), lambda i: (0, 0)),
        ],
        out_specs=pl.BlockSpec((_TM, 3 * H), lambda i: (i, 0)),
        out_shape=jax.ShapeDtypeStruct((M, 3 * H), jnp.float32),
        compiler_params=_PAR,
    )(obsT, dec_in_W.T, dec_in_b.reshape(1, H), dec_W_ih.T,
      dec_b_ih.reshape(1, 3 * H))

    logitsT, recon11 = pl.pallas_call(
        _dec_scan_kernel,
        grid=(S,),
        in_specs=[
            pl.BlockSpec((1, B, 3 * H), lambda t: (t, 0, 0)),
            pl.BlockSpec((1, B, A), lambda t: (t, 0, 0)),
            pl.BlockSpec((B, H), lambda t: (0, 0)),
            pl.BlockSpec((H, 3 * H), lambda t: (0, 0)),
            pl.BlockSpec((1, 3 * H), lambda t: (0, 0)),
            pl.BlockSpec((H, A), lambda t: (0, 0)),
            pl.BlockSpec((1, A), lambda t: (0, 0)),
        ],
        out_specs=[
            pl.BlockSpec((1, B, A), lambda t: (t, 0, 0)),
            pl.BlockSpec((1, 1), lambda t: (0, 0)),
        ],
        out_shape=[
            jax.ShapeDtypeStruct((S, B, A), jnp.float32),
            jax.ShapeDtypeStruct((1, 1), jnp.float32),
        ],
        scratch_shapes=[
            pltpu.VMEM((B, H), jnp.float32),
            pltpu.SMEM((1, 1), jnp.float32),
        ],
        compiler_params=_SEQ,
    )(gid.reshape(S, B, 3 * H), actT3, dh0, dec_W_hh.T,
      dec_b_hh.reshape(1, 3 * H), head_W.T, head_b.reshape(1, A))

    logits = jnp.swapaxes(logitsT, 0, 1)
    indices = idx8[:, 0]
    return (logits, indices, vq_loss11[0, 0], recon11[0, 0])


# speedup vs baseline: 2.4336x; 1.2632x over previous
import jax
import jax.numpy as jnp
from jax.experimental import pallas as pl
from jax.experimental.pallas import tpu as pltpu

_HID = 512
_K = 1024
_COMMIT = 0.25


def _mm(a, b):
    return jnp.dot(a, b, preferred_element_type=jnp.float32)


def _gru_step(gi, h, whh_ref, bhh_ref):
    gh = _mm(h, whh_ref[...]) + bhh_ref[...]
    i_r, i_z, i_n = gi[:, :_HID], gi[:, _HID:2 * _HID], gi[:, 2 * _HID:]
    h_r, h_z, h_n = gh[:, :_HID], gh[:, _HID:2 * _HID], gh[:, 2 * _HID:]
    r = jax.nn.sigmoid(i_r + h_r)
    z = jax.nn.sigmoid(i_z + h_z)
    n = jnp.tanh(i_n + r * h_n)
    return (1.0 - z) * n + z * h


def _enc_kernel(xin_ref, win_ref, b1_ref, wih0_ref, bih0_ref,
                whh0_ref, bhh0_ref, wih1_ref, bih1_ref, whh1_ref, bhh1_ref,
                wz_ref, bz_ref, cbT_ref, cb_ref, csq_ref, c2h_ref, c2hb_ref,
                idx_ref, loss_ref, dh0_ref, h0_ref, h1_ref):
    t = pl.program_id(0)

    @pl.when(t == 0)
    def _():
        h0_ref[...] = jnp.zeros_like(h0_ref)
        h1_ref[...] = jnp.zeros_like(h1_ref)

    x = jax.nn.relu(_mm(xin_ref[0], win_ref[...]) + b1_ref[...])
    gi0 = _mm(x, wih0_ref[...]) + bih0_ref[...]
    h0 = _gru_step(gi0, h0_ref[...], whh0_ref, bhh0_ref)
    h0_ref[...] = h0
    gi1 = _mm(h0, wih1_ref[...]) + bih1_ref[...]
    h1 = _gru_step(gi1, h1_ref[...], whh1_ref, bhh1_ref)
    h1_ref[...] = h1

    @pl.when(t == pl.num_programs(0) - 1)
    def _():
        z = _mm(h1, wz_ref[...]) + bz_ref[...]
        scores = _mm(z, cbT_ref[...])
        csq = csq_ref[...]
        zsq = jnp.sum(z * z, axis=1, keepdims=True)
        d = (zsq + csq) - 2.0 * scores
        m = jnp.min(d, axis=1, keepdims=True)
        iota = jax.lax.broadcasted_iota(jnp.int32, d.shape, 1)
        idx = jnp.min(jnp.where(d == m, iota, _K), axis=1, keepdims=True)
        idx_ref[...] = jnp.broadcast_to(idx, idx_ref.shape)
        onehot = (iota == idx).astype(jnp.float32)
        z_q = _mm(onehot, cb_ref[...])
        diff = z_q - z
        loss_ref[...] = ((1.0 + _COMMIT) * jnp.mean(diff * diff)).reshape(1, 1)
        dh0_ref[...] = jnp.tanh(_mm(z_q, c2h_ref[...]) + c2hb_ref[...])


def _dec_kernel(obs_ref, act_ref, dh0_ref, w1_ref, b1_ref, wih_ref, bih_ref,
                whh_ref, bhh_ref, whd_ref, bhd_ref,
                logits_ref, loss_ref, h_ref, acc_ref):
    t = pl.program_id(0)

    @pl.when(t == 0)
    def _():
        h_ref[...] = dh0_ref[...]
        acc_ref[0, 0] = 0.0

    x = jax.nn.relu(_mm(obs_ref[0], w1_ref[...]) + b1_ref[...])
    gi = _mm(x, wih_ref[...]) + bih_ref[...]
    h_new = _gru_step(gi, h_ref[...], whh_ref, bhh_ref)
    h_ref[...] = h_new
    logits = _mm(h_new, whd_ref[...]) + bhd_ref[...]
    logits_ref[0] = logits

    a = act_ref[0]
    na = a.shape[1]
    ai = jax.lax.broadcasted_iota(jnp.int32, a.shape, 1)
    amax = jnp.max(a, axis=1, keepdims=True)
    tgt = jnp.min(jnp.where(a == amax, ai, na), axis=1, keepdims=True)
    lmax = jnp.max(logits, axis=1, keepdims=True)
    lse = lmax + jnp.log(
        jnp.sum(jnp.exp(logits - lmax), axis=1, keepdims=True))
    li = jax.lax.broadcasted_iota(jnp.int32, logits.shape, 1)
    tgt_logp = jnp.sum(jnp.where(li == tgt, logits - lse, 0.0), axis=1)
    acc_ref[0, 0] += jnp.sum(tgt_logp)

    @pl.when(t == pl.num_programs(0) - 1)
    def _():
        denom = h_ref.shape[0] * pl.num_programs(0)
        loss_ref[...] = (-acc_ref[0, 0] / denom).reshape(1, 1)


_SEQ = pltpu.CompilerParams(dimension_semantics=("arbitrary",))


def kernel(obs_seq, act_seq, enc_in_W, enc_in_b, enc_W_ih0, enc_W_hh0,
           enc_b_ih0, enc_b_hh0, enc_W_ih1, enc_W_hh1, enc_b_ih1, enc_b_hh1,
           enc_out_W, enc_out_b, codebook, c2h_W, c2h_b, dec_in_W, dec_in_b,
           dec_W_ih, dec_W_hh, dec_b_ih, dec_b_hh, head_W, head_b):
    B, S, OBS = obs_seq.shape
    A = act_seq.shape[-1]
    H = _HID
    EMB = codebook.shape[1]

    obsT3 = jnp.swapaxes(obs_seq, 0, 1)
    actT3 = jnp.swapaxes(act_seq, 0, 1)
    xinT3 = jnp.concatenate([obsT3, actT3], axis=-1)
    csq = (codebook ** 2).sum(-1).reshape(1, _K)

    def _full(shape):
        return pl.BlockSpec(shape, lambda t: tuple(0 for _ in shape))

    idx8, vq_loss11, dh0 = pl.pallas_call(
        _enc_kernel,
        grid=(S,),
        in_specs=[
            pl.BlockSpec((1, B, OBS + A), lambda t: (t, 0, 0)),
            _full((OBS + A, H)), _full((1, H)),
            _full((H, 3 * H)), _full((1, 3 * H)),
            _full((H, 3 * H)), _full((1, 3 * H)),
            _full((H, 3 * H)), _full((1, 3 * H)),
            _full((H, 3 * H)), _full((1, 3 * H)),
            _full((H, EMB)), _full((1, EMB)),
            _full((EMB, _K)), _full((_K, EMB)), _full((1, _K)),
            _full((EMB, H)), _full((1, H)),
        ],
        out_specs=[
            pl.BlockSpec((B, 8), lambda t: (0, 0)),
            pl.BlockSpec((1, 1), lambda t: (0, 0)),
            pl.BlockSpec((B, H), lambda t: (0, 0)),
        ],
        out_shape=[
            jax.ShapeDtypeStruct((B, 8), jnp.int32),
            jax.ShapeDtypeStruct((1, 1), jnp.float32),
            jax.ShapeDtypeStruct((B, H), jnp.float32),
        ],
        scratch_shapes=[
            pltpu.VMEM((B, H), jnp.float32),
            pltpu.VMEM((B, H), jnp.float32),
        ],
        compiler_params=_SEQ,
    )(xinT3, enc_in_W.T,
      enc_in_b.reshape(1, H), enc_W_ih0.T, enc_b_ih0.reshape(1, 3 * H),
      enc_W_hh0.T, enc_b_hh0.reshape(1, 3 * H), enc_W_ih1.T,
      enc_b_ih1.reshape(1, 3 * H), enc_W_hh1.T, enc_b_hh1.reshape(1, 3 * H),
      enc_out_W.T, enc_out_b.reshape(1, EMB), codebook.T, codebook, csq,
      c2h_W.T, c2h_b.reshape(1, H))

    logitsT, recon11 = pl.pallas_call(
        _dec_kernel,
        grid=(S,),
        in_specs=[
            pl.BlockSpec((1, B, OBS), lambda t: (t, 0, 0)),
            pl.BlockSpec((1, B, A), lambda t: (t, 0, 0)),
            _full((B, H)),
            _full((OBS, H)), _full((1, H)),
            _full((H, 3 * H)), _full((1, 3 * H)),
            _full((H, 3 * H)), _full((1, 3 * H)),
            _full((H, A)), _full((1, A)),
        ],
        out_specs=[
            pl.BlockSpec((1, B, A), lambda t: (t, 0, 0)),
            pl.BlockSpec((1, 1), lambda t: (0, 0)),
        ],
        out_shape=[
            jax.ShapeDtypeStruct((S, B, A), jnp.float32),
            jax.ShapeDtypeStruct((1, 1), jnp.float32),
        ],
        scratch_shapes=[
            pltpu.VMEM((B, H), jnp.float32),
            pltpu.SMEM((1, 1), jnp.float32),
        ],
        compiler_params=_SEQ,
    )(obsT3, actT3, dh0, dec_in_W.T, dec_in_b.reshape(1, H), dec_W_ih.T,
      dec_b_ih.reshape(1, 3 * H), dec_W_hh.T, dec_b_hh.reshape(1, 3 * H),
      head_W.T, head_b.reshape(1, A))

    logits = jnp.swapaxes(logitsT, 0, 1)
    return (logits, idx8[:, 0], vq_loss11[0, 0], recon11[0, 0])
